# Initial kernel scaffold; baseline (speedup 1.0000x reference)
import jax
import jax.numpy as jnp
from jax.experimental import pallas as pl


def kernel(x, experts, gate_W, gate_b):
    raise NotImplementedError("write your pallas kernel here")


# speedup vs baseline: 15.0298x; 15.0298x over previous
import functools

import jax
import jax.numpy as jnp
from jax.experimental import pallas as pl


def _smoe_kernel(ex_ref, w_ref, b_ref, out_ref, *, H, W, E, K):
    P = H * W
    ex = ex_ref[0]
    pidx = jax.lax.broadcasted_iota(jnp.int32, (E, P), 1)
    wcol = pidx % W

    logits = jax.lax.dot(w_ref[4], ex, preferred_element_type=jnp.float32)
    for kh in range(3):
        for kw in range(3):
            sh, sw = kh - 1, kw - 1
            s = sh * W + sw
            if s == 0 and sw == 0:
                continue
            shifted = jnp.roll(ex, -s, axis=1)
            if s > 0:
                valid = pidx < (P - s)
            else:
                valid = pidx >= (-s)
            if sw == 1:
                valid = valid & (wcol != (W - 1))
            elif sw == -1:
                valid = valid & (wcol != 0)
            shifted = jnp.where(valid, shifted, 0.0)
            logits = logits + jax.lax.dot(
                w_ref[kh * 3 + kw], shifted, preferred_element_type=jnp.float32)
    logits = logits + b_ref[...]

    m = jnp.max(logits, axis=0, keepdims=True)
    e = jnp.exp(logits - m)
    ssum = jnp.sum(e, axis=0, keepdims=True)
    prod = ex * e / ssum

    iot = jax.lax.broadcasted_iota(jnp.int32, (E, P), 0)
    key = logits
    rows = []
    for _ in range(K):
        mj = jnp.max(key, axis=0, keepdims=True)
        cand = jnp.where(key == mj, iot, E)
        sel = jnp.min(cand, axis=0, keepdims=True)
        mask = iot == sel
        rows.append(jnp.sum(jnp.where(mask, prod, 0.0), axis=0, keepdims=True))
        key = jnp.where(mask, -1e30, key)
    out_ref[0] = jnp.concatenate(rows, axis=0)


def kernel(x, experts, gate_W, gate_b):
    del x
    N, E, H, W = experts.shape
    K = 8
    P = H * W
    ex = experts.reshape(N, E, P)
    w2 = jnp.transpose(gate_W, (2, 3, 0, 1)).reshape(9, E, E)
    b = gate_b.reshape(E, 1)
    out = pl.pallas_call(
        functools.partial(_smoe_kernel, H=H, W=W, E=E, K=K),
        grid=(N,),
        in_specs=[
            pl.BlockSpec((1, E, P), lambda n: (n, 0, 0)),
            pl.BlockSpec((9, E, E), lambda n: (0, 0, 0)),
            pl.BlockSpec((E, 1), lambda n: (0, 0)),
        ],
        out_specs=pl.BlockSpec((1, K, P), lambda n: (n, 0, 0)),
        out_shape=jax.ShapeDtypeStruct((N, K, P), jnp.float32),
    )(ex, w2, b)
    return out.reshape(N, K, H, W)
